# baseline (speedup 1.0000x reference)
import functools

import jax
import jax.numpy as jnp
from jax import lax
from jax.experimental import pallas as pl
from jax.experimental.pallas import tpu as pltpu
from jax.experimental.pallas import tpu_sc as plsc

HIDDEN = 4096
NPOP = 8
N_TOKENS = 8192
BT = 512


def _tc_body(x_ref, w_ref, b_ref, hidden_ref, scores_ref):
    xb = x_ref[...]
    hidden_ref[...] = xb
    scores_ref[...] = (
        lax.dot_general(
            xb, w_ref[...], (((1,), (1,)), ((), ())),
            preferred_element_type=jnp.float32,
        )
        + b_ref[...]
    )


def _tc_stage(x, W, b2d, interpret=False):
    return pl.pallas_call(
        _tc_body,
        grid=(N_TOKENS // BT,),
        in_specs=[
            pl.BlockSpec((BT, HIDDEN), lambda i: (i, 0)),
            pl.BlockSpec((NPOP, HIDDEN), lambda i: (0, 0)),
            pl.BlockSpec((1, NPOP), lambda i: (0, 0)),
        ],
        out_specs=[
            pl.BlockSpec((BT, HIDDEN), lambda i: (i, 0)),
            pl.BlockSpec((BT, NPOP), lambda i: (i, 0)),
        ],
        out_shape=[
            jax.ShapeDtypeStruct((N_TOKENS, HIDDEN), jnp.float32),
            jax.ShapeDtypeStruct((N_TOKENS, NPOP), jnp.float32),
        ],
        compiler_params=pltpu.CompilerParams(
            vmem_limit_bytes=100 * 1024 * 1024,
        ),
        interpret=interpret,
    )(x, W, b2d)


def _sc_dispatch(scores, pop_ids):
    mesh = plsc.VectorSubcoreMesh(core_axis_name="c", subcore_axis_name="s", num_cores=1, num_subcores=4)
    nc, ns = mesh.num_cores, mesh.num_subcores
    nw = nc * ns
    chunk = N_TOKENS // nw
    nl = 16

    @functools.partial(
        pl.kernel,
        out_type=jax.ShapeDtypeStruct((N_TOKENS,), jnp.float32),
        mesh=mesh,
        compiler_params=pltpu.CompilerParams(needs_layout_passes=False),
        scratch_types=[
            pltpu.VMEM((chunk,), jnp.int32),
            pltpu.VMEM((chunk * NPOP,), jnp.float32),
            pltpu.VMEM((chunk,), jnp.float32),
            pltpu.SemaphoreType.DMA,
            pltpu.SemaphoreType.DMA,
        ],
    )
    def gather_kernel(scores_hbm, ids_hbm, out_hbm, ids_v, sc_v, out_v,
                      sem_ids, sem_sc):
        wid = lax.axis_index("s") * nc + lax.axis_index("c")
        base = wid * chunk
        ids_cp = pltpu.async_copy(ids_hbm.at[pl.ds(base, chunk)], ids_v, sem_ids)
        sc_cp = pltpu.async_copy(
            scores_hbm.at[pl.ds(base * NPOP, chunk * NPOP)], sc_v, sem_sc)
        ids_cp.wait()
        sc_cp.wait()
        for i in range(chunk // nl):
            pop16 = ids_v[pl.ds(i * nl, nl)]
            row16 = lax.iota(jnp.int32, nl) + (i * nl)
            flat16 = row16 * NPOP + pop16
            out_v[pl.ds(i * nl, nl)] = plsc.load_gather(sc_v, [flat16])
        pltpu.sync_copy(out_v, out_hbm.at[pl.ds(base, chunk)])

    return gather_kernel(scores.reshape(N_TOKENS * NPOP), pop_ids)


def kernel(x, pop_ids, W, b):
    hidden, scores = _tc_stage(x, W, b.reshape(1, NPOP))
    values = _sc_dispatch(scores, pop_ids)
    return hidden, values.reshape(N_TOKENS, 1)


# speedup vs baseline: 1.0125x; 1.0125x over previous
import functools

import jax
import jax.numpy as jnp
from jax import lax
from jax.experimental import pallas as pl
from jax.experimental.pallas import tpu as pltpu
from jax.experimental.pallas import tpu_sc as plsc

HIDDEN = 4096
NPOP = 8
N_TOKENS = 8192
BT = 512


def _tc_body(x_ref, w_ref, b_ref, hidden_ref, scores_ref):
    xb = x_ref[...]
    hidden_ref[...] = xb
    scores_ref[...] = (
        lax.dot_general(
            xb, w_ref[...], (((1,), (1,)), ((), ())),
            preferred_element_type=jnp.float32,
        )
        + b_ref[...]
    )


def _tc_stage(x, W, b2d, interpret=False):
    return pl.pallas_call(
        _tc_body,
        grid=(N_TOKENS // BT,),
        in_specs=[
            pl.BlockSpec((BT, HIDDEN), lambda i: (i, 0)),
            pl.BlockSpec((NPOP, HIDDEN), lambda i: (0, 0)),
            pl.BlockSpec((1, NPOP), lambda i: (0, 0)),
        ],
        out_specs=[
            pl.BlockSpec((BT, HIDDEN), lambda i: (i, 0)),
            pl.BlockSpec((BT, NPOP), lambda i: (i, 0)),
        ],
        out_shape=[
            jax.ShapeDtypeStruct((N_TOKENS, HIDDEN), jnp.float32),
            jax.ShapeDtypeStruct((N_TOKENS, NPOP), jnp.float32),
        ],
        compiler_params=pltpu.CompilerParams(
            vmem_limit_bytes=100 * 1024 * 1024,
        ),
        interpret=interpret,
    )(x, W, b2d)


def _sc_dispatch(scores, pop_ids):
    mesh = plsc.VectorSubcoreMesh(core_axis_name="c", subcore_axis_name="s", num_cores=1)
    nc, ns = mesh.num_cores, mesh.num_subcores
    nw = nc * ns
    chunk = N_TOKENS // nw
    nl = 16

    @functools.partial(
        pl.kernel,
        out_type=jax.ShapeDtypeStruct((N_TOKENS,), jnp.float32),
        mesh=mesh,
        compiler_params=pltpu.CompilerParams(needs_layout_passes=False),
        scratch_types=[
            pltpu.VMEM((chunk,), jnp.int32),
            pltpu.VMEM((chunk * NPOP,), jnp.float32),
            pltpu.VMEM((chunk,), jnp.float32),
            pltpu.SemaphoreType.DMA,
            pltpu.SemaphoreType.DMA,
        ],
    )
    def gather_kernel(scores_hbm, ids_hbm, out_hbm, ids_v, sc_v, out_v,
                      sem_ids, sem_sc):
        wid = lax.axis_index("s") * nc + lax.axis_index("c")
        base = wid * chunk
        ids_cp = pltpu.async_copy(ids_hbm.at[pl.ds(base, chunk)], ids_v, sem_ids)
        sc_cp = pltpu.async_copy(
            scores_hbm.at[pl.ds(base * NPOP, chunk * NPOP)], sc_v, sem_sc)
        ids_cp.wait()
        sc_cp.wait()
        for i in range(chunk // nl):
            pop16 = ids_v[pl.ds(i * nl, nl)]
            row16 = lax.iota(jnp.int32, nl) + (i * nl)
            flat16 = row16 * NPOP + pop16
            out_v[pl.ds(i * nl, nl)] = plsc.load_gather(sc_v, [flat16])
        pltpu.sync_copy(out_v, out_hbm.at[pl.ds(base, chunk)])

    return gather_kernel(scores.reshape(N_TOKENS * NPOP), pop_ids)


def kernel(x, pop_ids, W, b):
    hidden, scores = _tc_stage(x, W, b.reshape(1, NPOP))
    values = _sc_dispatch(scores, pop_ids)
    return hidden, values.reshape(N_TOKENS, 1)
